# baseline (device time: 2129246 ns/iter reference)
import jax
import jax.numpy as jnp
from jax import lax
from jax.experimental import pallas as pl
from jax.experimental.pallas import tpu as pltpu

K = 16


def kernel(x):
    m, n = x.shape
    c = m // K

    def body(
        x_ref, out_ref, vsend, vrecv,
        load_sems, send_sems, recv_sems, drain_sems, credit_sem, copy_sem,
    ):
        my_x = lax.axis_index("x")
        my_y = lax.axis_index("y")
        my_z = lax.axis_index("z")
        peer_y = (my_x, 1 - my_y, my_z)

        barrier_sem = pltpu.get_barrier_semaphore()
        pl.semaphore_signal(
            barrier_sem, inc=1, device_id=peer_y,
            device_id_type=pl.DeviceIdType.MESH,
        )
        pl.semaphore_wait(barrier_sem, 1)

        local = pltpu.make_async_copy(
            x_ref, out_ref.at[pl.ds(my_y * m, m)], copy_sem
        )
        local.start()

        rdmas = []
        for k in range(K):
            slot = k % 2
            if k >= 2:
                pl.semaphore_wait(credit_sem, 1)
                rdmas[k - 2].wait_send()
            load = pltpu.make_async_copy(
                x_ref.at[pl.ds(k * c, c)], vsend.at[slot], load_sems.at[slot]
            )
            load.start()
            load.wait()
            rd = pltpu.make_async_remote_copy(
                src_ref=vsend.at[slot],
                dst_ref=vrecv.at[slot],
                send_sem=send_sems.at[slot],
                recv_sem=recv_sems.at[slot],
                device_id=peer_y,
                device_id_type=pl.DeviceIdType.MESH,
            )
            rd.start()
            rdmas.append(rd)

            rd.wait_recv()
            drain = pltpu.make_async_copy(
                vrecv.at[slot],
                out_ref.at[pl.ds((1 - my_y) * m + k * c, c)],
                drain_sems.at[slot],
            )
            drain.start()
            drain.wait()
            pl.semaphore_signal(
                credit_sem, inc=1, device_id=peer_y,
                device_id_type=pl.DeviceIdType.MESH,
            )

        rdmas[K - 2].wait_send()
        rdmas[K - 1].wait_send()
        pl.semaphore_wait(credit_sem, 2)
        local.wait()

    return pl.pallas_call(
        body,
        out_shape=jax.ShapeDtypeStruct((2 * m, n), x.dtype),
        in_specs=[pl.BlockSpec(memory_space=pltpu.MemorySpace.HBM)],
        out_specs=pl.BlockSpec(memory_space=pltpu.MemorySpace.HBM),
        scratch_shapes=[
            pltpu.VMEM((2, c, n), x.dtype),
            pltpu.VMEM((2, c, n), x.dtype),
            pltpu.SemaphoreType.DMA((2,)),
            pltpu.SemaphoreType.DMA((2,)),
            pltpu.SemaphoreType.DMA((2,)),
            pltpu.SemaphoreType.DMA((2,)),
            pltpu.SemaphoreType.REGULAR,
            pltpu.SemaphoreType.DMA,
        ],
        compiler_params=pltpu.CompilerParams(collective_id=0),
    )(x)


# device time: 502359 ns/iter; 4.2385x vs baseline; 4.2385x over previous
import jax
import jax.numpy as jnp
from jax import lax
from jax.experimental import pallas as pl
from jax.experimental.pallas import tpu as pltpu

K = 8


def kernel(x):
    m, n = x.shape
    h = m // 2
    c = h // K

    def body(x_ref, out_ref, ysend, yrecv, xrecv, lclbuf,
             y_send_sems, y_recv_sems, f_send_sems, f_recv_sems,
             load_sems, st_sems, dr_sems, drx_sems, lcl_ld_sems,
             lcl_st_sems, y_credit, x_credit):
        my_x = lax.axis_index("x")
        my_y = lax.axis_index("y")
        my_z = lax.axis_index("z")
        py = (my_x, 1 - my_y, my_z)
        px = (1 - my_x, my_y, my_z)

        half = my_x * h
        mine = my_y * m
        theirs = (1 - my_y) * m

        barrier_sem = pltpu.get_barrier_semaphore()
        for nbr in (py, px):
            pl.semaphore_signal(
                barrier_sem, inc=1, device_id=nbr,
                device_id_type=pl.DeviceIdType.MESH,
            )
        pl.semaphore_wait(barrier_sem, 2)

        y_rd, f_rd, dr, drx, st_my, lcl_st = {}, {}, {}, {}, {}, {}

        def make_y(k):
            return pltpu.make_async_remote_copy(
                src_ref=ysend.at[k % 2],
                dst_ref=yrecv.at[k % 4],
                send_sem=y_send_sems.at[k],
                recv_sem=y_recv_sems.at[k],
                device_id=py,
                device_id_type=pl.DeviceIdType.MESH,
            )

        def make_f(k):
            return pltpu.make_async_remote_copy(
                src_ref=yrecv.at[k % 4],
                dst_ref=xrecv.at[k % 2],
                send_sem=f_send_sems.at[k],
                recv_sem=f_recv_sems.at[k],
                device_id=px,
                device_id_type=pl.DeviceIdType.MESH,
            )

        def send_phase(k):
            s = k % 2
            if k >= 2:
                y_rd[k - 2].wait_send()
                st_my[k - 2].wait()
                lcl_st[k - 2].wait()
            if k >= 4:
                pl.semaphore_wait(y_credit, 1)
            ld = pltpu.make_async_copy(
                x_ref.at[pl.ds(half + k * c, c)], ysend.at[s],
                load_sems.at[s],
            )
            ld.start()
            ld.wait()
            y_rd[k] = make_y(k)
            y_rd[k].start()
            st_my[k] = pltpu.make_async_copy(
                ysend.at[s], out_ref.at[pl.ds(mine + half + k * c, c)],
                st_sems.at[s],
            )
            st_my[k].start()
            oth = (1 - my_x) * h + k * c
            lld = pltpu.make_async_copy(
                x_ref.at[pl.ds(oth, c)], lclbuf.at[s], lcl_ld_sems.at[s]
            )
            lld.start()
            lld.wait()
            lcl_st[k] = pltpu.make_async_copy(
                lclbuf.at[s], out_ref.at[pl.ds(mine + oth, c)],
                lcl_st_sems.at[s],
            )
            lcl_st[k].start()

        def recv_phase(j):
            if j >= 2:
                f_rd[j - 2].wait_send()
                dr[j - 2].wait()
                pl.semaphore_signal(
                    y_credit, inc=1, device_id=py,
                    device_id_type=pl.DeviceIdType.MESH,
                )
            y_rd[j].wait_recv()
            dr[j] = pltpu.make_async_copy(
                yrecv.at[j % 4],
                out_ref.at[pl.ds(theirs + half + j * c, c)],
                dr_sems.at[j % 2],
            )
            dr[j].start()
            if j >= 2:
                drx[j - 2].wait()
                pl.semaphore_signal(
                    x_credit, inc=1, device_id=px,
                    device_id_type=pl.DeviceIdType.MESH,
                )
                pl.semaphore_wait(x_credit, 1)
            f_rd[j] = make_f(j)
            f_rd[j].start()
            if j >= 1:
                f_rd[j - 1].wait_recv()
                oth = (1 - my_x) * h + (j - 1) * c
                drx[j - 1] = pltpu.make_async_copy(
                    xrecv.at[(j - 1) % 2],
                    out_ref.at[pl.ds(theirs + oth, c)],
                    drx_sems.at[(j - 1) % 2],
                )
                drx[j - 1].start()

        for k in range(K):
            send_phase(k)
            if k >= 1:
                recv_phase(k - 1)
        recv_phase(K - 1)

        f_rd[K - 1].wait_recv()
        oth = (1 - my_x) * h + (K - 1) * c
        drx[K - 1] = pltpu.make_async_copy(
            xrecv.at[(K - 1) % 2],
            out_ref.at[pl.ds(theirs + oth, c)],
            drx_sems.at[(K - 1) % 2],
        )
        drx[K - 1].start()
        for j in (K - 2, K - 1):
            drx[j].wait()
            pl.semaphore_signal(
                x_credit, inc=1, device_id=px,
                device_id_type=pl.DeviceIdType.MESH,
            )
            f_rd[j].wait_send()
            dr[j].wait()
            pl.semaphore_signal(
                y_credit, inc=1, device_id=py,
                device_id_type=pl.DeviceIdType.MESH,
            )
            y_rd[j].wait_send()
            st_my[j].wait()
            lcl_st[j].wait()
        pl.semaphore_wait(y_credit, 4)
        pl.semaphore_wait(x_credit, 2)

    return pl.pallas_call(
        body,
        out_shape=jax.ShapeDtypeStruct((2 * m, n), x.dtype),
        in_specs=[pl.BlockSpec(memory_space=pltpu.MemorySpace.HBM)],
        out_specs=pl.BlockSpec(memory_space=pltpu.MemorySpace.HBM),
        scratch_shapes=[
            pltpu.VMEM((2, c, n), x.dtype),
            pltpu.VMEM((4, c, n), x.dtype),
            pltpu.VMEM((2, c, n), x.dtype),
            pltpu.VMEM((2, c, n), x.dtype),
            pltpu.SemaphoreType.DMA((K,)),
            pltpu.SemaphoreType.DMA((K,)),
            pltpu.SemaphoreType.DMA((K,)),
            pltpu.SemaphoreType.DMA((K,)),
            pltpu.SemaphoreType.DMA((2,)),
            pltpu.SemaphoreType.DMA((2,)),
            pltpu.SemaphoreType.DMA((2,)),
            pltpu.SemaphoreType.DMA((2,)),
            pltpu.SemaphoreType.DMA((2,)),
            pltpu.SemaphoreType.DMA((2,)),
            pltpu.SemaphoreType.REGULAR,
            pltpu.SemaphoreType.REGULAR,
        ],
        compiler_params=pltpu.CompilerParams(
            collective_id=0, vmem_limit_bytes=48 * 1024 * 1024
        ),
    )(x)


# device time: 480880 ns/iter; 4.4278x vs baseline; 1.0447x over previous
import jax
import jax.numpy as jnp
from jax import lax
from jax.experimental import pallas as pl
from jax.experimental.pallas import tpu as pltpu

K = 16


def kernel(x):
    m, n = x.shape
    h = m // 2
    c = h // K

    def body(x_ref, out_ref, ysend, yrecv, xrecv, lclbuf,
             y_send_sems, y_recv_sems, f_send_sems, f_recv_sems,
             load_sems, st_sems, dr_sems, drx_sems, lcl_ld_sems,
             lcl_st_sems, y_credit, x_credit):
        my_x = lax.axis_index("x")
        my_y = lax.axis_index("y")
        my_z = lax.axis_index("z")
        py = (my_x, 1 - my_y, my_z)
        px = (1 - my_x, my_y, my_z)

        half = my_x * h
        mine = my_y * m
        theirs = (1 - my_y) * m

        barrier_sem = pltpu.get_barrier_semaphore()
        for nbr in (py, px):
            pl.semaphore_signal(
                barrier_sem, inc=1, device_id=nbr,
                device_id_type=pl.DeviceIdType.MESH,
            )
        pl.semaphore_wait(barrier_sem, 2)

        y_rd, f_rd, dr, drx, st_my, lcl_st = {}, {}, {}, {}, {}, {}

        def make_y(k):
            return pltpu.make_async_remote_copy(
                src_ref=ysend.at[k % 2],
                dst_ref=yrecv.at[k % 4],
                send_sem=y_send_sems.at[k],
                recv_sem=y_recv_sems.at[k],
                device_id=py,
                device_id_type=pl.DeviceIdType.MESH,
            )

        def make_f(k):
            return pltpu.make_async_remote_copy(
                src_ref=yrecv.at[k % 4],
                dst_ref=xrecv.at[k % 2],
                send_sem=f_send_sems.at[k],
                recv_sem=f_recv_sems.at[k],
                device_id=px,
                device_id_type=pl.DeviceIdType.MESH,
            )

        def send_phase(k):
            s = k % 2
            if k >= 2:
                y_rd[k - 2].wait_send()
                st_my[k - 2].wait()
                lcl_st[k - 2].wait()
            if k >= 4:
                pl.semaphore_wait(y_credit, 1)
            ld = pltpu.make_async_copy(
                x_ref.at[pl.ds(half + k * c, c)], ysend.at[s],
                load_sems.at[s],
            )
            ld.start()
            ld.wait()
            y_rd[k] = make_y(k)
            y_rd[k].start()
            st_my[k] = pltpu.make_async_copy(
                ysend.at[s], out_ref.at[pl.ds(mine + half + k * c, c)],
                st_sems.at[s],
            )
            st_my[k].start()
            oth = (1 - my_x) * h + k * c
            lld = pltpu.make_async_copy(
                x_ref.at[pl.ds(oth, c)], lclbuf.at[s], lcl_ld_sems.at[s]
            )
            lld.start()
            lld.wait()
            lcl_st[k] = pltpu.make_async_copy(
                lclbuf.at[s], out_ref.at[pl.ds(mine + oth, c)],
                lcl_st_sems.at[s],
            )
            lcl_st[k].start()

        def recv_phase(j):
            if j >= 2:
                f_rd[j - 2].wait_send()
                dr[j - 2].wait()
                pl.semaphore_signal(
                    y_credit, inc=1, device_id=py,
                    device_id_type=pl.DeviceIdType.MESH,
                )
            y_rd[j].wait_recv()
            dr[j] = pltpu.make_async_copy(
                yrecv.at[j % 4],
                out_ref.at[pl.ds(theirs + half + j * c, c)],
                dr_sems.at[j % 2],
            )
            dr[j].start()
            if j >= 2:
                drx[j - 2].wait()
                pl.semaphore_signal(
                    x_credit, inc=1, device_id=px,
                    device_id_type=pl.DeviceIdType.MESH,
                )
                pl.semaphore_wait(x_credit, 1)
            f_rd[j] = make_f(j)
            f_rd[j].start()
            if j >= 1:
                f_rd[j - 1].wait_recv()
                oth = (1 - my_x) * h + (j - 1) * c
                drx[j - 1] = pltpu.make_async_copy(
                    xrecv.at[(j - 1) % 2],
                    out_ref.at[pl.ds(theirs + oth, c)],
                    drx_sems.at[(j - 1) % 2],
                )
                drx[j - 1].start()

        for k in range(K):
            send_phase(k)
            if k >= 1:
                recv_phase(k - 1)
        recv_phase(K - 1)

        f_rd[K - 1].wait_recv()
        oth = (1 - my_x) * h + (K - 1) * c
        drx[K - 1] = pltpu.make_async_copy(
            xrecv.at[(K - 1) % 2],
            out_ref.at[pl.ds(theirs + oth, c)],
            drx_sems.at[(K - 1) % 2],
        )
        drx[K - 1].start()
        for j in (K - 2, K - 1):
            drx[j].wait()
            pl.semaphore_signal(
                x_credit, inc=1, device_id=px,
                device_id_type=pl.DeviceIdType.MESH,
            )
            f_rd[j].wait_send()
            dr[j].wait()
            pl.semaphore_signal(
                y_credit, inc=1, device_id=py,
                device_id_type=pl.DeviceIdType.MESH,
            )
            y_rd[j].wait_send()
            st_my[j].wait()
            lcl_st[j].wait()
        pl.semaphore_wait(y_credit, 4)
        pl.semaphore_wait(x_credit, 2)

    return pl.pallas_call(
        body,
        out_shape=jax.ShapeDtypeStruct((2 * m, n), x.dtype),
        in_specs=[pl.BlockSpec(memory_space=pltpu.MemorySpace.HBM)],
        out_specs=pl.BlockSpec(memory_space=pltpu.MemorySpace.HBM),
        scratch_shapes=[
            pltpu.VMEM((2, c, n), x.dtype),
            pltpu.VMEM((4, c, n), x.dtype),
            pltpu.VMEM((2, c, n), x.dtype),
            pltpu.VMEM((2, c, n), x.dtype),
            pltpu.SemaphoreType.DMA((K,)),
            pltpu.SemaphoreType.DMA((K,)),
            pltpu.SemaphoreType.DMA((K,)),
            pltpu.SemaphoreType.DMA((K,)),
            pltpu.SemaphoreType.DMA((2,)),
            pltpu.SemaphoreType.DMA((2,)),
            pltpu.SemaphoreType.DMA((2,)),
            pltpu.SemaphoreType.DMA((2,)),
            pltpu.SemaphoreType.DMA((2,)),
            pltpu.SemaphoreType.DMA((2,)),
            pltpu.SemaphoreType.REGULAR,
            pltpu.SemaphoreType.REGULAR,
        ],
        compiler_params=pltpu.CompilerParams(
            collective_id=0, vmem_limit_bytes=48 * 1024 * 1024
        ),
    )(x)


# device time: 392783 ns/iter; 5.4209x vs baseline; 1.2243x over previous
import jax
import jax.numpy as jnp
from jax import lax
from jax.experimental import pallas as pl
from jax.experimental.pallas import tpu as pltpu

KQ = 8


def kernel(x):
    m, n = x.shape
    q = m // 4
    c = q // KQ

    def body(x_ref, out_ref, ysend, yrecv, xrecv, zrecv, xrecv2, zrecv2,
             lclbuf, y_s, y_r, xf1_s, xf1_r, zf1_s, zf1_r, xf2_s, xf2_r,
             zf2_s, zf2_r, ld_sems, st_sems, dry_sems, drx1_sems, drz1_sems,
             drx2_sems, drz2_sems, lcl_ld_sems, lcl_st_sems,
             y_cred, x1_cred, z1_cred, x2_cred, z2_cred):
        my_x = lax.axis_index("x")
        my_y = lax.axis_index("y")
        my_z = lax.axis_index("z")
        z1 = my_z % 2
        py = (my_x, 1 - my_y, my_z)
        px = (1 - my_x, my_y, my_z)
        pz = (my_x, my_y, my_z + 1 - 2 * z1)

        myq = my_x * (2 * q) + z1 * q
        xq = (1 - my_x) * (2 * q) + z1 * q
        zq = my_x * (2 * q) + (1 - z1) * q
        dq = (1 - my_x) * (2 * q) + (1 - z1) * q
        mine = my_y * m
        theirs = (1 - my_y) * m

        barrier_sem = pltpu.get_barrier_semaphore()
        for nbr in (py, px, pz):
            pl.semaphore_signal(
                barrier_sem, inc=1, device_id=nbr,
                device_id_type=pl.DeviceIdType.MESH,
            )
        pl.semaphore_wait(barrier_sem, 3)

        def remote(src, dst, ss, rs, dev):
            return pltpu.make_async_remote_copy(
                src_ref=src, dst_ref=dst, send_sem=ss, recv_sem=rs,
                device_id=dev, device_id_type=pl.DeviceIdType.MESH,
            )

        def r_y(k):
            return remote(ysend.at[k % 2], yrecv.at[k % 4],
                          y_s.at[k], y_r.at[k], py)

        def r_xf1(k):
            return remote(yrecv.at[k % 4], xrecv.at[k % 4],
                          xf1_s.at[k], xf1_r.at[k], px)

        def r_zf1(k):
            return remote(yrecv.at[k % 4], zrecv.at[k % 4],
                          zf1_s.at[k], zf1_r.at[k], pz)

        def r_xf2(e):
            return remote(zrecv.at[(2 * e) % 4], xrecv2.at[e % 2],
                          xf2_s.at[e], xf2_r.at[e], px)

        def r_zf2(o):
            return remote(xrecv.at[(2 * o + 1) % 4], zrecv2.at[o % 2],
                          zf2_s.at[o], zf2_r.at[o], pz)

        st_my, dry_d, drx1_d, drz1_d, drx2_d, drz2_d, lcl_st_d = (
            {}, {}, {}, {}, {}, {}, {}
        )

        def sig(sem, dev):
            pl.semaphore_signal(sem, inc=1, device_id=dev,
                                device_id_type=pl.DeviceIdType.MESH)

        def send_phase(k):
            s = k % 2
            if k >= 2:
                r_y(k - 2).wait_send()
                st_my[k - 2].wait()
            if k >= 4:
                pl.semaphore_wait(y_cred, 1)
            ld = pltpu.make_async_copy(
                x_ref.at[pl.ds(myq + k * c, c)], ysend.at[s], ld_sems.at[s]
            )
            ld.start()
            ld.wait()
            r_y(k).start()
            st_my[k] = pltpu.make_async_copy(
                ysend.at[s], out_ref.at[pl.ds(mine + myq + k * c, c)],
                st_sems.at[s],
            )
            st_my[k].start()

        def yrecv_phase(j):
            if j >= 2:
                r_xf1(j - 2).wait_send()
                r_zf1(j - 2).wait_send()
                dry_d[j - 2].wait()
                sig(y_cred, py)
            r_y(j).wait_recv()
            dry_d[j] = pltpu.make_async_copy(
                yrecv.at[j % 4],
                out_ref.at[pl.ds(theirs + myq + j * c, c)],
                dry_sems.at[j % 2],
            )
            dry_d[j].start()
            if j >= 4:
                pl.semaphore_wait(x1_cred, 1)
            r_xf1(j).start()
            if j >= 4:
                pl.semaphore_wait(z1_cred, 1)
            r_zf1(j).start()

        def xz1_phase(i):
            if i >= 2:
                c2 = i - 2
                drx1_d[c2].wait()
                if c2 % 2 == 1:
                    r_zf2((c2 - 1) // 2).wait_send()
                sig(x1_cred, px)
                drz1_d[c2].wait()
                if c2 % 2 == 0:
                    r_xf2(c2 // 2).wait_send()
                sig(z1_cred, pz)
            r_xf1(i).wait_recv()
            drx1_d[i] = pltpu.make_async_copy(
                xrecv.at[i % 4],
                out_ref.at[pl.ds(theirs + xq + i * c, c)],
                drx1_sems.at[i % 2],
            )
            drx1_d[i].start()
            r_zf1(i).wait_recv()
            drz1_d[i] = pltpu.make_async_copy(
                zrecv.at[i % 4],
                out_ref.at[pl.ds(theirs + zq + i * c, c)],
                drz1_sems.at[i % 2],
            )
            drz1_d[i].start()
            if i % 2 == 0:
                e = i // 2
                if e >= 2:
                    pl.semaphore_wait(x2_cred, 1)
                r_xf2(e).start()
            else:
                o = (i - 1) // 2
                if o >= 2:
                    pl.semaphore_wait(z2_cred, 1)
                r_zf2(o).start()

        def diag_phase(d):
            if d % 2 == 0:
                e = d // 2
                if e >= 1:
                    drx2_d[e - 1].wait()
                    sig(x2_cred, px)
                r_xf2(e).wait_recv()
                drx2_d[e] = pltpu.make_async_copy(
                    xrecv2.at[e % 2],
                    out_ref.at[pl.ds(theirs + dq + d * c, c)],
                    drx2_sems.at[e % 2],
                )
                drx2_d[e].start()
            else:
                o = (d - 1) // 2
                if o >= 1:
                    drz2_d[o - 1].wait()
                    sig(z2_cred, pz)
                r_zf2(o).wait_recv()
                drz2_d[o] = pltpu.make_async_copy(
                    zrecv2.at[o % 2],
                    out_ref.at[pl.ds(theirs + dq + d * c, c)],
                    drz2_sems.at[o % 2],
                )
                drz2_d[o].start()

        def lcl_phase(k):
            for t, qoff in enumerate([xq, zq, dq]):
                idx = 3 * k + t
                s = idx % 2
                if idx >= 2:
                    lcl_st_d[idx - 2].wait()
                ld = pltpu.make_async_copy(
                    x_ref.at[pl.ds(qoff + k * c, c)], lclbuf.at[s],
                    lcl_ld_sems.at[s],
                )
                ld.start()
                ld.wait()
                lcl_st_d[idx] = pltpu.make_async_copy(
                    lclbuf.at[s],
                    out_ref.at[pl.ds(mine + qoff + k * c, c)],
                    lcl_st_sems.at[s],
                )
                lcl_st_d[idx].start()

        for k in range(KQ):
            send_phase(k)
            if k >= 1:
                yrecv_phase(k - 1)
            if k >= 2:
                xz1_phase(k - 2)
            if k >= 3:
                diag_phase(k - 3)
            lcl_phase(k)

        yrecv_phase(KQ - 1)
        xz1_phase(KQ - 2)
        diag_phase(KQ - 3)
        xz1_phase(KQ - 1)
        diag_phase(KQ - 2)
        diag_phase(KQ - 1)

        for j in (KQ - 2, KQ - 1):
            r_y(j).wait_send()
            st_my[j].wait()
            r_xf1(j).wait_send()
            r_zf1(j).wait_send()
            dry_d[j].wait()
            sig(y_cred, py)
            drx1_d[j].wait()
            drz1_d[j].wait()
        r_zf2(3).wait_send()
        r_xf2(3).wait_send()
        sig(x1_cred, px)
        sig(x1_cred, px)
        sig(z1_cred, pz)
        sig(z1_cred, pz)
        drx2_d[3].wait()
        sig(x2_cred, px)
        drz2_d[3].wait()
        sig(z2_cred, pz)
        lcl_st_d[3 * KQ - 2].wait()
        lcl_st_d[3 * KQ - 1].wait()
        pl.semaphore_wait(y_cred, 4)
        pl.semaphore_wait(x1_cred, 4)
        pl.semaphore_wait(z1_cred, 4)
        pl.semaphore_wait(x2_cred, 2)
        pl.semaphore_wait(z2_cred, 2)

    return pl.pallas_call(
        body,
        out_shape=jax.ShapeDtypeStruct((2 * m, n), x.dtype),
        in_specs=[pl.BlockSpec(memory_space=pltpu.MemorySpace.HBM)],
        out_specs=pl.BlockSpec(memory_space=pltpu.MemorySpace.HBM),
        scratch_shapes=[
            pltpu.VMEM((2, c, n), x.dtype),
            pltpu.VMEM((4, c, n), x.dtype),
            pltpu.VMEM((4, c, n), x.dtype),
            pltpu.VMEM((4, c, n), x.dtype),
            pltpu.VMEM((2, c, n), x.dtype),
            pltpu.VMEM((2, c, n), x.dtype),
            pltpu.VMEM((2, c, n), x.dtype),
            pltpu.SemaphoreType.DMA((KQ,)),
            pltpu.SemaphoreType.DMA((KQ,)),
            pltpu.SemaphoreType.DMA((KQ,)),
            pltpu.SemaphoreType.DMA((KQ,)),
            pltpu.SemaphoreType.DMA((KQ,)),
            pltpu.SemaphoreType.DMA((KQ,)),
            pltpu.SemaphoreType.DMA((KQ // 2,)),
            pltpu.SemaphoreType.DMA((KQ // 2,)),
            pltpu.SemaphoreType.DMA((KQ // 2,)),
            pltpu.SemaphoreType.DMA((KQ // 2,)),
            pltpu.SemaphoreType.DMA((2,)),
            pltpu.SemaphoreType.DMA((2,)),
            pltpu.SemaphoreType.DMA((2,)),
            pltpu.SemaphoreType.DMA((2,)),
            pltpu.SemaphoreType.DMA((2,)),
            pltpu.SemaphoreType.DMA((2,)),
            pltpu.SemaphoreType.DMA((2,)),
            pltpu.SemaphoreType.DMA((2,)),
            pltpu.SemaphoreType.DMA((2,)),
            pltpu.SemaphoreType.REGULAR,
            pltpu.SemaphoreType.REGULAR,
            pltpu.SemaphoreType.REGULAR,
            pltpu.SemaphoreType.REGULAR,
            pltpu.SemaphoreType.REGULAR,
        ],
        compiler_params=pltpu.CompilerParams(
            collective_id=0, vmem_limit_bytes=48 * 1024 * 1024
        ),
    )(x)


# device time: 381622 ns/iter; 5.5795x vs baseline; 1.0292x over previous
import jax
import jax.numpy as jnp
from jax import lax
from jax.experimental import pallas as pl
from jax.experimental.pallas import tpu as pltpu

KQ = 16


def kernel(x):
    m, n = x.shape
    q = m // 4
    c = q // KQ

    def body(x_ref, out_ref, ysend, yrecv, xrecv, zrecv, xrecv2, zrecv2,
             lclbuf, y_s, y_r, xf1_s, xf1_r, zf1_s, zf1_r, xf2_s, xf2_r,
             zf2_s, zf2_r, ld_sems, st_sems, dry_sems, drx1_sems, drz1_sems,
             drx2_sems, drz2_sems, lcl_ld_sems, lcl_st_sems,
             y_cred, x1_cred, z1_cred, x2_cred, z2_cred):
        my_x = lax.axis_index("x")
        my_y = lax.axis_index("y")
        my_z = lax.axis_index("z")
        z1 = my_z % 2
        py = (my_x, 1 - my_y, my_z)
        px = (1 - my_x, my_y, my_z)
        pz = (my_x, my_y, my_z + 1 - 2 * z1)

        myq = my_x * (2 * q) + z1 * q
        xq = (1 - my_x) * (2 * q) + z1 * q
        zq = my_x * (2 * q) + (1 - z1) * q
        dq = (1 - my_x) * (2 * q) + (1 - z1) * q
        mine = my_y * m
        theirs = (1 - my_y) * m

        barrier_sem = pltpu.get_barrier_semaphore()
        for nbr in (py, px, pz):
            pl.semaphore_signal(
                barrier_sem, inc=1, device_id=nbr,
                device_id_type=pl.DeviceIdType.MESH,
            )
        pl.semaphore_wait(barrier_sem, 3)

        def remote(src, dst, ss, rs, dev):
            return pltpu.make_async_remote_copy(
                src_ref=src, dst_ref=dst, send_sem=ss, recv_sem=rs,
                device_id=dev, device_id_type=pl.DeviceIdType.MESH,
            )

        def r_y(k):
            return remote(ysend.at[k % 2], yrecv.at[k % 4],
                          y_s.at[k], y_r.at[k], py)

        def r_xf1(k):
            return remote(yrecv.at[k % 4], xrecv.at[k % 4],
                          xf1_s.at[k], xf1_r.at[k], px)

        def r_zf1(k):
            return remote(yrecv.at[k % 4], zrecv.at[k % 4],
                          zf1_s.at[k], zf1_r.at[k], pz)

        def r_xf2(e):
            return remote(zrecv.at[(2 * e) % 4], xrecv2.at[e % 2],
                          xf2_s.at[e], xf2_r.at[e], px)

        def r_zf2(o):
            return remote(xrecv.at[(2 * o + 1) % 4], zrecv2.at[o % 2],
                          zf2_s.at[o], zf2_r.at[o], pz)

        st_my, dry_d, drx1_d, drz1_d, drx2_d, drz2_d, lcl_st_d = (
            {}, {}, {}, {}, {}, {}, {}
        )

        def sig(sem, dev):
            pl.semaphore_signal(sem, inc=1, device_id=dev,
                                device_id_type=pl.DeviceIdType.MESH)

        def send_phase(k):
            s = k % 2
            if k >= 2:
                r_y(k - 2).wait_send()
                st_my[k - 2].wait()
            if k >= 4:
                pl.semaphore_wait(y_cred, 1)
            ld = pltpu.make_async_copy(
                x_ref.at[pl.ds(myq + k * c, c)], ysend.at[s], ld_sems.at[s]
            )
            ld.start()
            ld.wait()
            r_y(k).start()
            st_my[k] = pltpu.make_async_copy(
                ysend.at[s], out_ref.at[pl.ds(mine + myq + k * c, c)],
                st_sems.at[s],
            )
            st_my[k].start()

        def yrecv_phase(j):
            if j >= 2:
                r_xf1(j - 2).wait_send()
                r_zf1(j - 2).wait_send()
                dry_d[j - 2].wait()
                sig(y_cred, py)
            r_y(j).wait_recv()
            dry_d[j] = pltpu.make_async_copy(
                yrecv.at[j % 4],
                out_ref.at[pl.ds(theirs + myq + j * c, c)],
                dry_sems.at[j % 2],
            )
            dry_d[j].start()
            if j >= 4:
                pl.semaphore_wait(x1_cred, 1)
            r_xf1(j).start()
            if j >= 4:
                pl.semaphore_wait(z1_cred, 1)
            r_zf1(j).start()

        def xz1_phase(i):
            if i >= 2:
                c2 = i - 2
                drx1_d[c2].wait()
                if c2 % 2 == 1:
                    r_zf2((c2 - 1) // 2).wait_send()
                sig(x1_cred, px)
                drz1_d[c2].wait()
                if c2 % 2 == 0:
                    r_xf2(c2 // 2).wait_send()
                sig(z1_cred, pz)
            r_xf1(i).wait_recv()
            drx1_d[i] = pltpu.make_async_copy(
                xrecv.at[i % 4],
                out_ref.at[pl.ds(theirs + xq + i * c, c)],
                drx1_sems.at[i % 2],
            )
            drx1_d[i].start()
            r_zf1(i).wait_recv()
            drz1_d[i] = pltpu.make_async_copy(
                zrecv.at[i % 4],
                out_ref.at[pl.ds(theirs + zq + i * c, c)],
                drz1_sems.at[i % 2],
            )
            drz1_d[i].start()
            if i % 2 == 0:
                e = i // 2
                if e >= 2:
                    pl.semaphore_wait(x2_cred, 1)
                r_xf2(e).start()
            else:
                o = (i - 1) // 2
                if o >= 2:
                    pl.semaphore_wait(z2_cred, 1)
                r_zf2(o).start()

        def diag_phase(d):
            if d % 2 == 0:
                e = d // 2
                if e >= 1:
                    drx2_d[e - 1].wait()
                    sig(x2_cred, px)
                r_xf2(e).wait_recv()
                drx2_d[e] = pltpu.make_async_copy(
                    xrecv2.at[e % 2],
                    out_ref.at[pl.ds(theirs + dq + d * c, c)],
                    drx2_sems.at[e % 2],
                )
                drx2_d[e].start()
            else:
                o = (d - 1) // 2
                if o >= 1:
                    drz2_d[o - 1].wait()
                    sig(z2_cred, pz)
                r_zf2(o).wait_recv()
                drz2_d[o] = pltpu.make_async_copy(
                    zrecv2.at[o % 2],
                    out_ref.at[pl.ds(theirs + dq + d * c, c)],
                    drz2_sems.at[o % 2],
                )
                drz2_d[o].start()

        def lcl_phase(k):
            for t, qoff in enumerate([xq, zq, dq]):
                idx = 3 * k + t
                s = idx % 2
                if idx >= 2:
                    lcl_st_d[idx - 2].wait()
                ld = pltpu.make_async_copy(
                    x_ref.at[pl.ds(qoff + k * c, c)], lclbuf.at[s],
                    lcl_ld_sems.at[s],
                )
                ld.start()
                ld.wait()
                lcl_st_d[idx] = pltpu.make_async_copy(
                    lclbuf.at[s],
                    out_ref.at[pl.ds(mine + qoff + k * c, c)],
                    lcl_st_sems.at[s],
                )
                lcl_st_d[idx].start()

        for k in range(KQ):
            send_phase(k)
            if k >= 1:
                yrecv_phase(k - 1)
            if k >= 2:
                xz1_phase(k - 2)
            if k >= 3:
                diag_phase(k - 3)
            lcl_phase(k)

        yrecv_phase(KQ - 1)
        xz1_phase(KQ - 2)
        diag_phase(KQ - 3)
        xz1_phase(KQ - 1)
        diag_phase(KQ - 2)
        diag_phase(KQ - 1)

        for j in (KQ - 2, KQ - 1):
            r_y(j).wait_send()
            st_my[j].wait()
            r_xf1(j).wait_send()
            r_zf1(j).wait_send()
            dry_d[j].wait()
            sig(y_cred, py)
            drx1_d[j].wait()
            drz1_d[j].wait()
        r_zf2(KQ // 2 - 1).wait_send()
        r_xf2(KQ // 2 - 1).wait_send()
        sig(x1_cred, px)
        sig(x1_cred, px)
        sig(z1_cred, pz)
        sig(z1_cred, pz)
        drx2_d[KQ // 2 - 1].wait()
        sig(x2_cred, px)
        drz2_d[KQ // 2 - 1].wait()
        sig(z2_cred, pz)
        lcl_st_d[3 * KQ - 2].wait()
        lcl_st_d[3 * KQ - 1].wait()
        pl.semaphore_wait(y_cred, 4)
        pl.semaphore_wait(x1_cred, 4)
        pl.semaphore_wait(z1_cred, 4)
        pl.semaphore_wait(x2_cred, 2)
        pl.semaphore_wait(z2_cred, 2)

    return pl.pallas_call(
        body,
        out_shape=jax.ShapeDtypeStruct((2 * m, n), x.dtype),
        in_specs=[pl.BlockSpec(memory_space=pltpu.MemorySpace.HBM)],
        out_specs=pl.BlockSpec(memory_space=pltpu.MemorySpace.HBM),
        scratch_shapes=[
            pltpu.VMEM((2, c, n), x.dtype),
            pltpu.VMEM((4, c, n), x.dtype),
            pltpu.VMEM((4, c, n), x.dtype),
            pltpu.VMEM((4, c, n), x.dtype),
            pltpu.VMEM((2, c, n), x.dtype),
            pltpu.VMEM((2, c, n), x.dtype),
            pltpu.VMEM((2, c, n), x.dtype),
            pltpu.SemaphoreType.DMA((KQ,)),
            pltpu.SemaphoreType.DMA((KQ,)),
            pltpu.SemaphoreType.DMA((KQ,)),
            pltpu.SemaphoreType.DMA((KQ,)),
            pltpu.SemaphoreType.DMA((KQ,)),
            pltpu.SemaphoreType.DMA((KQ,)),
            pltpu.SemaphoreType.DMA((KQ // 2,)),
            pltpu.SemaphoreType.DMA((KQ // 2,)),
            pltpu.SemaphoreType.DMA((KQ // 2,)),
            pltpu.SemaphoreType.DMA((KQ // 2,)),
            pltpu.SemaphoreType.DMA((2,)),
            pltpu.SemaphoreType.DMA((2,)),
            pltpu.SemaphoreType.DMA((2,)),
            pltpu.SemaphoreType.DMA((2,)),
            pltpu.SemaphoreType.DMA((2,)),
            pltpu.SemaphoreType.DMA((2,)),
            pltpu.SemaphoreType.DMA((2,)),
            pltpu.SemaphoreType.DMA((2,)),
            pltpu.SemaphoreType.DMA((2,)),
            pltpu.SemaphoreType.REGULAR,
            pltpu.SemaphoreType.REGULAR,
            pltpu.SemaphoreType.REGULAR,
            pltpu.SemaphoreType.REGULAR,
            pltpu.SemaphoreType.REGULAR,
        ],
        compiler_params=pltpu.CompilerParams(
            collective_id=0, vmem_limit_bytes=48 * 1024 * 1024
        ),
    )(x)
